# Initial kernel scaffold; baseline (speedup 1.0000x reference)
import jax
import jax.numpy as jnp
from jax.experimental import pallas as pl


def kernel(x, graph, ne_W1, ne_b1, ne_W2, ne_b2, ee_W1, ee_b1, ee_W2, ee_b2, cn_W1, cn_b1, cn_W2, cn_b2, ce_W1, ce_b1, ce_W2, ce_b2, out_W1, out_b1, out_W2, out_b2):
    raise NotImplementedError("write your pallas kernel here")


# speedup vs baseline: 1.6147x; 1.6147x over previous
import functools

import jax
import jax.numpy as jnp
from jax import lax
from jax.experimental import pallas as pl
from jax.experimental.pallas import tpu as pltpu
from jax.experimental.pallas import tpu_sc as plsc

N = 50000
E = 800000
ITERS = 8
F = 64
NC, NS = 2, 16
NW = NC * NS
HALF = N // 2
TRASH = HALF
ACC_R = 25024
RPT = ACC_R // NS
NB_G = (2 * E) // 128
NB_S = E // 128


def _silu(x):
    return x / (1.0 + jnp.exp(-x))


def _full(shape):
    return pl.BlockSpec(shape, lambda i: (0,) * len(shape))


def _enc_pre(x8, nw1, nb1, nw2, nb2, wa, wb):
    B = 2000

    def body(x_ref, nw1_r, nb1_r, nw2_r, nb2_r, wa_r, wb_r, n_out, a_out, b_out):
        xv = x_ref[...]
        h = _silu(xv @ nw1_r[...] + nb1_r[...])
        n_out[...] = _silu(h @ nw2_r[...] + nb2_r[...])
        a_out[...] = xv @ wa_r[...]
        b_out[...] = xv @ wb_r[...]

    return pl.pallas_call(
        body,
        grid=(N // B,),
        in_specs=[pl.BlockSpec((B, 8), lambda i: (i, 0)),
                  _full((8, F)), _full((1, F)), _full((F, F)), _full((1, F)),
                  _full((8, F)), _full((8, F))],
        out_specs=[pl.BlockSpec((B, F), lambda i: (i, 0))] * 3,
        out_shape=[jax.ShapeDtypeStruct((N, F), jnp.float32)] * 3,
    )(x8, nw1, nb1, nw2, nb2, wa, wb)


def _edge_enc(g, w2, b1, b2):
    B = 2000
    nblk = E // B

    def body(ga_ref, gb_ref, w2_r, b1_r, b2_r, out):
        h = _silu(ga_ref[...] + gb_ref[...] + b1_r[...])
        out[...] = _silu(h @ w2_r[...] + b2_r[...])

    return pl.pallas_call(
        body,
        grid=(nblk,),
        in_specs=[pl.BlockSpec((B, F), lambda i: (i, 0)),
                  pl.BlockSpec((B, F), lambda i: (i + nblk, 0)),
                  _full((F, F)), _full((1, F)), _full((1, F))],
        out_specs=pl.BlockSpec((B, F), lambda i: (i, 0)),
        out_shape=jax.ShapeDtypeStruct((E, F), jnp.float32),
    )(g, g, w2, b1, b2)


def _node_update(nodes, msg2, w1a, w1b, b1, w2, b2):
    B = 1000
    nhalf = HALF // B

    def body(n_ref, m_ref, w1a_r, w1b_r, b1_r, w2_r, b2_r, out):
        n = n_ref[...]
        m = m_ref[0]
        h = _silu(n @ w1a_r[...] + m @ w1b_r[...] + b1_r[...])
        out[...] = _silu(h @ w2_r[...] + b2_r[...]) + n

    return pl.pallas_call(
        body,
        grid=(N // B,),
        in_specs=[pl.BlockSpec((B, F), lambda i: (i, 0)),
                  pl.BlockSpec((1, B, F), lambda i: (i // nhalf, i % nhalf, 0)),
                  _full((F, F)), _full((F, F)), _full((1, F)),
                  _full((F, F)), _full((1, F))],
        out_specs=pl.BlockSpec((B, F), lambda i: (i, 0)),
        out_shape=jax.ShapeDtypeStruct((N, F), jnp.float32),
    )(nodes, msg2, w1a, w1b, b1, w2, b2)


def _edge_update(g, edges, wa, wb, wc, b1, w2, b2):
    B = 2000
    nblk = E // B

    def body(ga_ref, gb_ref, e_ref, wa_r, wb_r, wc_r, b1_r, w2_r, b2_r, out):
        e = e_ref[...]
        h = _silu(ga_ref[...] @ wa_r[...] + gb_ref[...] @ wb_r[...]
                  + e @ wc_r[...] + b1_r[...])
        out[...] = jnp.tanh(h @ w2_r[...] + b2_r[...]) + e

    return pl.pallas_call(
        body,
        grid=(nblk,),
        in_specs=[pl.BlockSpec((B, F), lambda i: (i, 0)),
                  pl.BlockSpec((B, F), lambda i: (i + nblk, 0)),
                  pl.BlockSpec((B, F), lambda i: (i, 0)),
                  _full((F, F)), _full((F, F)), _full((F, F)), _full((1, F)),
                  _full((F, F)), _full((1, F))],
        out_specs=pl.BlockSpec((B, F), lambda i: (i, 0)),
        out_shape=jax.ShapeDtypeStruct((E, F), jnp.float32),
    )(g, g, edges, wa, wb, wc, b1, w2, b2)


def _head(nodes, w1, b1, w2p, b2p):
    B = 2000

    def body(n_ref, w1_r, b1_r, w2_r, b2_r, out):
        h = _silu(n_ref[...] @ w1_r[...] + b1_r[...])
        e = h @ w2_r[...] + b2_r[...]
        nrm = jnp.sqrt(jnp.sum(e * e, axis=1, keepdims=True))
        out[...] = e / jnp.maximum(nrm, 1e-12)

    return pl.pallas_call(
        body,
        grid=(N // B,),
        in_specs=[pl.BlockSpec((B, F), lambda i: (i, 0)),
                  _full((F, F)), _full((1, F)), _full((F, 128)), _full((1, 128))],
        out_specs=pl.BlockSpec((B, 128), lambda i: (i, 0)),
        out_shape=jax.ShapeDtypeStruct((N, 128), jnp.float32),
    )(nodes, w1, b1, w2p, b2p)


def _make_gather(T):
    mesh = plsc.VectorSubcoreMesh(core_axis_name="c", subcore_axis_name="s")
    base_n = NB_G // NW
    extra = NB_G - base_n * NW

    @functools.partial(
        pl.kernel,
        out_type=jax.ShapeDtypeStruct((2 * E, F), jnp.float32),
        mesh=mesh,
        compiler_params=pltpu.CompilerParams(use_tc_tiling_on_sc=False),
        scratch_types=[pltpu.VMEM((1, 128), jnp.int32),
                       pltpu.VMEM((128, F), jnp.float32)],
    )
    def k(tbl_hbm, idx_hbm, out_hbm, ibuf, rows):
        c = lax.axis_index("c")
        s = lax.axis_index("s")
        wid = s * NC + c
        nb = base_n + jnp.where(wid < extra, 1, 0)
        start = wid * base_n + jnp.minimum(wid, extra)

        def body(kk, carry):
            r = start + kk
            pltpu.sync_copy(idx_hbm.at[pl.ds(r, 1)], ibuf)
            pltpu.sync_copy(tbl_hbm.at[ibuf.at[0]], rows)
            pltpu.sync_copy(rows, out_hbm.at[pl.ds(r * 128, 128)])
            return carry

        lax.fori_loop(0, nb, body, 0)

    return k


def _make_scatter():
    mesh = plsc.VectorSubcoreMesh(core_axis_name="c", subcore_axis_name="s")
    base_n = NB_S // NS
    extra = NB_S - base_n * NS
    nfull = RPT // 128
    rem = RPT - nfull * 128

    @functools.partial(
        pl.kernel,
        out_type=jax.ShapeDtypeStruct((2 * ACC_R, F), jnp.float32),
        mesh=mesh,
        compiler_params=pltpu.CompilerParams(use_tc_tiling_on_sc=False),
        scratch_types=[pltpu.VMEM((1, 128), jnp.int32),
                       pltpu.VMEM((128, F), jnp.float32),
                       pltpu.VMEM((128, F), jnp.float32),
                       pltpu.VMEM_SHARED((ACC_R, F), jnp.float32)],
    )
    def k(e_hbm, idx_hbm, zeros_hbm, out_hbm, ibuf, ebuf, stage, acc):
        c = lax.axis_index("c")
        s = lax.axis_index("s")
        r0 = s * RPT
        pltpu.sync_copy(zeros_hbm, stage)

        def zb(j, carry):
            pltpu.sync_copy(stage, acc.at[pl.ds(r0 + j * 128, 128)])
            return carry

        lax.fori_loop(0, nfull, zb, 0)
        pltpu.sync_copy(stage.at[pl.ds(0, rem)],
                        acc.at[pl.ds(r0 + nfull * 128, rem)])
        plsc.subcore_barrier()

        nb = base_n + jnp.where(s < extra, 1, 0)
        start = s * base_n + jnp.minimum(s, extra)

        def body(kk, carry):
            r = start + kk
            pltpu.sync_copy(idx_hbm.at[pl.ds(c * NB_S + r, 1)], ibuf)
            pltpu.sync_copy(e_hbm.at[pl.ds(r * 128, 128)], ebuf)
            pltpu.sync_copy(ebuf, acc.at[ibuf.at[0]], add=True)
            return carry

        lax.fori_loop(0, nb, body, 0)
        plsc.subcore_barrier()

        def rb(j, carry):
            rr = r0 + j * 128
            pltpu.sync_copy(acc.at[pl.ds(rr, 128)], stage)
            pltpu.sync_copy(stage, out_hbm.at[pl.ds(c * ACC_R + rr, 128)])
            return carry

        lax.fori_loop(0, nfull, rb, 0)
        rr = r0 + nfull * 128
        pltpu.sync_copy(acc.at[pl.ds(rr, rem)], stage.at[pl.ds(0, rem)])
        pltpu.sync_copy(stage.at[pl.ds(0, rem)],
                        out_hbm.at[pl.ds(c * ACC_R + rr, rem)])

    return k


_gather_nodes = _make_gather(N)
_gather_enc = _make_gather(2 * N)
_scatter = _make_scatter()


def kernel(x, graph, ne_W1, ne_b1, ne_W2, ne_b2, ee_W1, ee_b1, ee_W2, ee_b2,
           cn_W1, cn_b1, cn_W2, cn_b2, ce_W1, ce_b1, ce_W2, ce_b2,
           out_W1, out_b1, out_W2, out_b2):
    src = graph[0]
    dst = graph[1]

    x8 = jnp.pad(x, ((0, 0), (0, 5)))
    nw1p = jnp.pad(ne_W1, ((0, 5), (0, 0)))
    wa = jnp.pad(ee_W1[:3], ((0, 5), (0, 0)))
    wb = jnp.pad(ee_W1[3:], ((0, 5), (0, 0)))
    r1 = lambda b: b.reshape(1, -1)

    idx_enc = jnp.concatenate([src, dst + N]).reshape(NB_G, 128)
    idx_it = jnp.concatenate([src, dst]).reshape(NB_G, 128)
    loc0 = jnp.where(dst < HALF, dst, TRASH)
    loc1 = jnp.where(dst >= HALF, dst - HALF, TRASH)
    idx_sc = jnp.concatenate([loc0, loc1]).reshape(2 * NB_S, 128)
    zeros128 = jnp.zeros((128, F), jnp.float32)

    w1a, w1b = cn_W1[:F], cn_W1[F:]
    ewa, ewb, ewc = ce_W1[:F], ce_W1[F:2 * F], ce_W1[2 * F:]
    w2p = jnp.pad(out_W2, ((0, 0), (0, 128 - 12)))
    b2p = jnp.pad(out_b2, (0, 128 - 12)).reshape(1, 128)

    nodes, xa, xb = _enc_pre(x8, nw1p, r1(ne_b1), ne_W2, r1(ne_b2), wa, wb)
    g0 = _gather_enc(jnp.concatenate([xa, xb], axis=0), idx_enc)
    edges = _edge_enc(g0, ee_W2, r1(ee_b1), r1(ee_b2))

    for _ in range(ITERS):
        msg2 = _scatter(edges, idx_sc, zeros128).reshape(2, ACC_R, F)
        nodes = _node_update(nodes, msg2, w1a, w1b, r1(cn_b1), cn_W2, r1(cn_b2))
        g = _gather_nodes(nodes, idx_it)
        edges = _edge_update(g, edges, ewa, ewb, ewc, r1(ce_b1), ce_W2, r1(ce_b2))

    embp = _head(nodes, out_W1, r1(out_b1), w2p, b2p)
    return (embp[:, :12], nodes, edges)


# speedup vs baseline: 1.8816x; 1.1653x over previous
import functools

import jax
import jax.numpy as jnp
from jax import lax
from jax.experimental import pallas as pl
from jax.experimental.pallas import tpu as pltpu
from jax.experimental.pallas import tpu_sc as plsc

N = 50000
E = 800000
ITERS = 8
F = 64
NC, NS = 2, 16
NW = NC * NS
HALF = N // 2
TRASH = HALF
ACC_R = 25024
RPT = ACC_R // NS
NB_G = (2 * E) // 128
NB_S = E // 128


def _silu(x):
    return x / (1.0 + jnp.exp(-x))


def _full(shape):
    return pl.BlockSpec(shape, lambda i: (0,) * len(shape))


def _enc_pre(x8, nw1, nb1, nw2, nb2, wa, wb):
    B = 2000

    def body(x_ref, nw1_r, nb1_r, nw2_r, nb2_r, wa_r, wb_r, n_out, a_out, b_out):
        xv = x_ref[...]
        h = _silu(xv @ nw1_r[...] + nb1_r[...])
        n_out[...] = _silu(h @ nw2_r[...] + nb2_r[...])
        a_out[...] = xv @ wa_r[...]
        b_out[...] = xv @ wb_r[...]

    return pl.pallas_call(
        body,
        grid=(N // B,),
        in_specs=[pl.BlockSpec((B, 8), lambda i: (i, 0)),
                  _full((8, F)), _full((1, F)), _full((F, F)), _full((1, F)),
                  _full((8, F)), _full((8, F))],
        out_specs=[pl.BlockSpec((B, F), lambda i: (i, 0))] * 3,
        out_shape=[jax.ShapeDtypeStruct((N, F), jnp.float32)] * 3,
    )(x8, nw1, nb1, nw2, nb2, wa, wb)


def _edge_enc(g, w2, b1, b2):
    B = 2000
    nblk = E // B

    def body(ga_ref, gb_ref, w2_r, b1_r, b2_r, out):
        h = _silu(ga_ref[...] + gb_ref[...] + b1_r[...])
        out[...] = _silu(h @ w2_r[...] + b2_r[...])

    return pl.pallas_call(
        body,
        grid=(nblk,),
        in_specs=[pl.BlockSpec((B, F), lambda i: (i, 0)),
                  pl.BlockSpec((B, F), lambda i: (i + nblk, 0)),
                  _full((F, F)), _full((1, F)), _full((1, F))],
        out_specs=pl.BlockSpec((B, F), lambda i: (i, 0)),
        out_shape=jax.ShapeDtypeStruct((E, F), jnp.float32),
    )(g, g, w2, b1, b2)


def _node_update(nodes, msg2, w1a, w1b, b1, w2, b2):
    B = 1000
    nhalf = HALF // B

    def body(n_ref, m_ref, w1a_r, w1b_r, b1_r, w2_r, b2_r, out):
        n = n_ref[...]
        m = m_ref[0]
        h = _silu(n @ w1a_r[...] + m @ w1b_r[...] + b1_r[...])
        out[...] = _silu(h @ w2_r[...] + b2_r[...]) + n

    return pl.pallas_call(
        body,
        grid=(N // B,),
        in_specs=[pl.BlockSpec((B, F), lambda i: (i, 0)),
                  pl.BlockSpec((1, B, F), lambda i: (i // nhalf, i % nhalf, 0)),
                  _full((F, F)), _full((F, F)), _full((1, F)),
                  _full((F, F)), _full((1, F))],
        out_specs=pl.BlockSpec((B, F), lambda i: (i, 0)),
        out_shape=jax.ShapeDtypeStruct((N, F), jnp.float32),
    )(nodes, msg2, w1a, w1b, b1, w2, b2)


def _edge_update(g, edges, wa, wb, wc, b1, w2, b2):
    B = 2000
    nblk = E // B

    def body(ga_ref, gb_ref, e_ref, wa_r, wb_r, wc_r, b1_r, w2_r, b2_r, out):
        e = e_ref[...]
        h = _silu(ga_ref[...] @ wa_r[...] + gb_ref[...] @ wb_r[...]
                  + e @ wc_r[...] + b1_r[...])
        out[...] = jnp.tanh(h @ w2_r[...] + b2_r[...]) + e

    return pl.pallas_call(
        body,
        grid=(nblk,),
        in_specs=[pl.BlockSpec((B, F), lambda i: (i, 0)),
                  pl.BlockSpec((B, F), lambda i: (i + nblk, 0)),
                  pl.BlockSpec((B, F), lambda i: (i, 0)),
                  _full((F, F)), _full((F, F)), _full((F, F)), _full((1, F)),
                  _full((F, F)), _full((1, F))],
        out_specs=pl.BlockSpec((B, F), lambda i: (i, 0)),
        out_shape=jax.ShapeDtypeStruct((E, F), jnp.float32),
    )(g, g, edges, wa, wb, wc, b1, w2, b2)


def _head(nodes, w1, b1, w2p, b2p):
    B = 2000

    def body(n_ref, w1_r, b1_r, w2_r, b2_r, out):
        h = _silu(n_ref[...] @ w1_r[...] + b1_r[...])
        e = h @ w2_r[...] + b2_r[...]
        nrm = jnp.sqrt(jnp.sum(e * e, axis=1, keepdims=True))
        out[...] = e / jnp.maximum(nrm, 1e-12)

    return pl.pallas_call(
        body,
        grid=(N // B,),
        in_specs=[pl.BlockSpec((B, F), lambda i: (i, 0)),
                  _full((F, F)), _full((1, F)), _full((F, 128)), _full((1, 128))],
        out_specs=pl.BlockSpec((B, 128), lambda i: (i, 0)),
        out_shape=jax.ShapeDtypeStruct((N, 128), jnp.float32),
    )(nodes, w1, b1, w2p, b2p)


NB_GP = 12544
GPT = NB_GP // 8 // NW


def _make_gather(T):
    mesh = plsc.VectorSubcoreMesh(core_axis_name="c", subcore_axis_name="s")

    @functools.partial(
        pl.kernel,
        out_type=jax.ShapeDtypeStruct((NB_GP * 128, F), jnp.float32),
        mesh=mesh,
        compiler_params=pltpu.CompilerParams(use_tc_tiling_on_sc=False),
        scratch_types=[pltpu.VMEM((8, 128), jnp.int32)]
                      + [pltpu.VMEM((128, F), jnp.float32)] * 8
                      + [pltpu.SemaphoreType.DMA, pltpu.SemaphoreType.DMA],
    )
    def k(tbl_hbm, idx_hbm, out_hbm, ib, r0, r1, r2, r3, r4, r5, r6, r7,
          semg, semw):
        c = lax.axis_index("c")
        s = lax.axis_index("s")
        wid = s * NC + c
        rs = [r0, r1, r2, r3, r4, r5, r6, r7]

        def body(q, carry):
            c0 = (wid * GPT + q) * 8
            pltpu.sync_copy(idx_hbm.at[pl.ds(c0, 8)], ib)

            @pl.when(q > 0)
            def _():
                for j in range(8):
                    pltpu.make_async_copy(out_hbm.at[pl.ds(0, 128)],
                                          rs[j], semw).wait()

            hs = [pltpu.async_copy(tbl_hbm.at[ib.at[j]], rs[j], semg)
                  for j in range(8)]
            for h in hs:
                h.wait()
            for j in range(8):
                pltpu.async_copy(rs[j], out_hbm.at[pl.ds((c0 + j) * 128, 128)],
                                 semw)
            return carry

        lax.fori_loop(0, GPT, body, 0)
        for j in range(8):
            pltpu.make_async_copy(out_hbm.at[pl.ds(0, 128)], rs[j], semw).wait()

    return k


def _make_scatter():
    mesh = plsc.VectorSubcoreMesh(core_axis_name="c", subcore_axis_name="s")
    base_n = NB_S // NS
    extra = NB_S - base_n * NS
    nfull = RPT // 128
    rem = RPT - nfull * 128

    npair = NB_S // 2
    base_p = npair // NS
    extra_p = npair - base_p * NS

    @functools.partial(
        pl.kernel,
        out_type=jax.ShapeDtypeStruct((2 * ACC_R, F), jnp.float32),
        mesh=mesh,
        compiler_params=pltpu.CompilerParams(use_tc_tiling_on_sc=False),
        scratch_types=[pltpu.VMEM((2, 128), jnp.int32),
                       pltpu.VMEM((128, F), jnp.float32),
                       pltpu.VMEM((128, F), jnp.float32)]
                      + [pltpu.SemaphoreType.DMA] * 3
                      + [pltpu.VMEM_SHARED((ACC_R, F), jnp.float32)],
    )
    def k(e_hbm, idx_hbm, zeros_hbm, out_hbm, ib, b0, b1,
          seme, sema, semz, acc):
        c = lax.axis_index("c")
        s = lax.axis_index("s")
        r0 = s * RPT
        pltpu.sync_copy(zeros_hbm, b0)
        hz = [pltpu.async_copy(b0, acc.at[pl.ds(r0 + j * 128, 128)], semz)
              for j in range(nfull)]
        hz.append(pltpu.async_copy(b0.at[pl.ds(0, rem)],
                                   acc.at[pl.ds(r0 + nfull * 128, rem)], semz))
        for h in hz:
            h.wait()
        plsc.subcore_barrier()

        np_t = base_p + jnp.where(s < extra_p, 1, 0)
        start = s * base_p + jnp.minimum(s, extra_p)

        def body(kk, carry):
            r = (start + kk) * 2

            @pl.when(kk > 0)
            def _():
                pltpu.make_async_copy(zeros_hbm, b0, sema).wait()
                pltpu.make_async_copy(zeros_hbm, b1, sema).wait()

            pltpu.sync_copy(idx_hbm.at[pl.ds(c * NB_S + r, 2)], ib)
            h0 = pltpu.async_copy(e_hbm.at[pl.ds(r * 128, 128)], b0, seme)
            h1 = pltpu.async_copy(e_hbm.at[pl.ds((r + 1) * 128, 128)], b1, semz)
            h0.wait()
            pltpu.async_copy(b0, acc.at[ib.at[0]], sema, add=True)
            h1.wait()
            pltpu.async_copy(b1, acc.at[ib.at[1]], sema, add=True)
            return carry

        lax.fori_loop(0, np_t, body, 0)
        pltpu.make_async_copy(zeros_hbm, b0, sema).wait()
        pltpu.make_async_copy(zeros_hbm, b1, sema).wait()
        plsc.subcore_barrier()

        for g in range(6):
            h0 = pltpu.async_copy(acc.at[pl.ds(r0 + (2 * g) * 128, 128)],
                                  b0, seme)
            h1 = pltpu.async_copy(acc.at[pl.ds(r0 + (2 * g + 1) * 128, 128)],
                                  b1, semz)
            h0.wait()
            w0 = pltpu.async_copy(
                b0, out_hbm.at[pl.ds(c * ACC_R + r0 + (2 * g) * 128, 128)],
                sema)
            h1.wait()
            w1 = pltpu.async_copy(
                b1, out_hbm.at[pl.ds(c * ACC_R + r0 + (2 * g + 1) * 128, 128)],
                sema)
            w0.wait()
            w1.wait()
        rr = r0 + nfull * 128
        pltpu.sync_copy(acc.at[pl.ds(rr, rem)], b0.at[pl.ds(0, rem)])
        pltpu.sync_copy(b0.at[pl.ds(0, rem)],
                        out_hbm.at[pl.ds(c * ACC_R + rr, rem)])

    return k


_gather_nodes = _make_gather(N)
_gather_enc = _make_gather(2 * N)
_scatter = _make_scatter()


def kernel(x, graph, ne_W1, ne_b1, ne_W2, ne_b2, ee_W1, ee_b1, ee_W2, ee_b2,
           cn_W1, cn_b1, cn_W2, cn_b2, ce_W1, ce_b1, ce_W2, ce_b2,
           out_W1, out_b1, out_W2, out_b2):
    src = graph[0]
    dst = graph[1]

    x8 = jnp.pad(x, ((0, 0), (0, 5)))
    nw1p = jnp.pad(ne_W1, ((0, 5), (0, 0)))
    wa = jnp.pad(ee_W1[:3], ((0, 5), (0, 0)))
    wb = jnp.pad(ee_W1[3:], ((0, 5), (0, 0)))
    r1 = lambda b: b.reshape(1, -1)

    pad_g = ((0, NB_GP - NB_G), (0, 0))
    idx_enc = jnp.pad(jnp.concatenate([src, dst + N]).reshape(NB_G, 128), pad_g)
    idx_it = jnp.pad(jnp.concatenate([src, dst]).reshape(NB_G, 128), pad_g)
    loc0 = jnp.where(dst < HALF, dst, TRASH)
    loc1 = jnp.where(dst >= HALF, dst - HALF, TRASH)
    idx_sc = jnp.concatenate([loc0, loc1]).reshape(2 * NB_S, 128)
    zeros128 = jnp.zeros((128, F), jnp.float32)

    w1a, w1b = cn_W1[:F], cn_W1[F:]
    ewa, ewb, ewc = ce_W1[:F], ce_W1[F:2 * F], ce_W1[2 * F:]
    w2p = jnp.pad(out_W2, ((0, 0), (0, 128 - 12)))
    b2p = jnp.pad(out_b2, (0, 128 - 12)).reshape(1, 128)

    nodes, xa, xb = _enc_pre(x8, nw1p, r1(ne_b1), ne_W2, r1(ne_b2), wa, wb)
    g0 = _gather_enc(jnp.concatenate([xa, xb], axis=0), idx_enc)
    edges = _edge_enc(g0, ee_W2, r1(ee_b1), r1(ee_b2))

    for _ in range(ITERS):
        msg2 = _scatter(edges, idx_sc, zeros128).reshape(2, ACC_R, F)
        nodes = _node_update(nodes, msg2, w1a, w1b, r1(cn_b1), cn_W2, r1(cn_b2))
        g = _gather_nodes(nodes, idx_it)
        edges = _edge_update(g, edges, ewa, ewb, ewc, r1(ce_b1), ce_W2, r1(ce_b2))

    embp = _head(nodes, out_W1, r1(out_b1), w2p, b2p)
    return (embp[:, :12], nodes, edges)


# speedup vs baseline: 2.0265x; 1.0770x over previous
import functools

import jax
import jax.numpy as jnp
from jax import lax
from jax.experimental import pallas as pl
from jax.experimental.pallas import tpu as pltpu
from jax.experimental.pallas import tpu_sc as plsc

N = 50000
E = 800000
ITERS = 8
F = 64
NC, NS = 2, 16
NW = NC * NS
HALF = N // 2
TRASH = HALF
ACC_R = 25024
RPT = ACC_R // NS
NB_G = (2 * E) // 128
NB_S = E // 128


def _silu(x):
    return x / (1.0 + jnp.exp(-x))


def _full(shape):
    return pl.BlockSpec(shape, lambda i: (0,) * len(shape))


def _enc_pre(x8, nw1, nb1, nw2, nb2, wa, wb):
    B = 10000

    def body(x_ref, nw1_r, nb1_r, nw2_r, nb2_r, wa_r, wb_r, n_out, a_out, b_out):
        xv = x_ref[...]
        h = _silu(xv @ nw1_r[...] + nb1_r[...])
        n_out[...] = _silu(h @ nw2_r[...] + nb2_r[...])
        a_out[...] = xv @ wa_r[...]
        b_out[...] = xv @ wb_r[...]

    return pl.pallas_call(
        body,
        grid=(N // B,),
        in_specs=[pl.BlockSpec((B, 8), lambda i: (i, 0)),
                  _full((8, F)), _full((1, F)), _full((F, F)), _full((1, F)),
                  _full((8, F)), _full((8, F))],
        out_specs=[pl.BlockSpec((B, F), lambda i: (i, 0))] * 3,
        out_shape=[jax.ShapeDtypeStruct((N, F), jnp.float32)] * 3,
    )(x8, nw1, nb1, nw2, nb2, wa, wb)


def _edge_enc(g, w2, b1, b2):
    B = 8000
    nblk = E // B

    def body(ga_ref, gb_ref, w2_r, b1_r, b2_r, out):
        h = _silu(ga_ref[...] + gb_ref[...] + b1_r[...])
        out[...] = _silu(h @ w2_r[...] + b2_r[...])

    return pl.pallas_call(
        body,
        grid=(nblk,),
        in_specs=[pl.BlockSpec((B, F), lambda i: (i, 0)),
                  pl.BlockSpec((B, F), lambda i: (i + nblk, 0)),
                  _full((F, F)), _full((1, F)), _full((1, F))],
        out_specs=pl.BlockSpec((B, F), lambda i: (i, 0)),
        out_shape=jax.ShapeDtypeStruct((E, F), jnp.float32),
    )(g, g, w2, b1, b2)


def _node_update(nodes, msg2, w1a, w1b, b1, w2, b2):
    B = 5000
    nhalf = HALF // B

    def body(n_ref, m_ref, w1a_r, w1b_r, b1_r, w2_r, b2_r, out):
        n = n_ref[...]
        m = m_ref[0]
        h = _silu(n @ w1a_r[...] + m @ w1b_r[...] + b1_r[...])
        out[...] = _silu(h @ w2_r[...] + b2_r[...]) + n

    return pl.pallas_call(
        body,
        grid=(N // B,),
        in_specs=[pl.BlockSpec((B, F), lambda i: (i, 0)),
                  pl.BlockSpec((1, B, F), lambda i: (i // nhalf, i % nhalf, 0)),
                  _full((F, F)), _full((F, F)), _full((1, F)),
                  _full((F, F)), _full((1, F))],
        out_specs=pl.BlockSpec((B, F), lambda i: (i, 0)),
        out_shape=jax.ShapeDtypeStruct((N, F), jnp.float32),
    )(nodes, msg2, w1a, w1b, b1, w2, b2)


def _edge_update(g, edges, wa, wb, wc, b1, w2, b2):
    B = 8000
    nblk = E // B

    def body(ga_ref, gb_ref, e_ref, wa_r, wb_r, wc_r, b1_r, w2_r, b2_r, out):
        e = e_ref[...]
        h = _silu(ga_ref[...] @ wa_r[...] + gb_ref[...] @ wb_r[...]
                  + e @ wc_r[...] + b1_r[...])
        out[...] = jnp.tanh(h @ w2_r[...] + b2_r[...]) + e

    return pl.pallas_call(
        body,
        grid=(nblk,),
        in_specs=[pl.BlockSpec((B, F), lambda i: (i, 0)),
                  pl.BlockSpec((B, F), lambda i: (i + nblk, 0)),
                  pl.BlockSpec((B, F), lambda i: (i, 0)),
                  _full((F, F)), _full((F, F)), _full((F, F)), _full((1, F)),
                  _full((F, F)), _full((1, F))],
        out_specs=pl.BlockSpec((B, F), lambda i: (i, 0)),
        out_shape=jax.ShapeDtypeStruct((E, F), jnp.float32),
    )(g, g, edges, wa, wb, wc, b1, w2, b2)


def _head(nodes, w1, b1, w2p, b2p):
    B = 10000

    def body(n_ref, w1_r, b1_r, w2_r, b2_r, out):
        h = _silu(n_ref[...] @ w1_r[...] + b1_r[...])
        e = h @ w2_r[...] + b2_r[...]
        nrm = jnp.sqrt(jnp.sum(e * e, axis=1, keepdims=True))
        out[...] = e / jnp.maximum(nrm, 1e-12)

    return pl.pallas_call(
        body,
        grid=(N // B,),
        in_specs=[pl.BlockSpec((B, F), lambda i: (i, 0)),
                  _full((F, F)), _full((1, F)), _full((F, 128)), _full((1, 128))],
        out_specs=pl.BlockSpec((B, 128), lambda i: (i, 0)),
        out_shape=jax.ShapeDtypeStruct((N, 128), jnp.float32),
    )(nodes, w1, b1, w2p, b2p)


NB_GP = 12544
GPT = NB_GP // 8 // NW


def _make_gather(T):
    mesh = plsc.VectorSubcoreMesh(core_axis_name="c", subcore_axis_name="s")

    @functools.partial(
        pl.kernel,
        out_type=jax.ShapeDtypeStruct((NB_GP * 128, F), jnp.float32),
        mesh=mesh,
        compiler_params=pltpu.CompilerParams(use_tc_tiling_on_sc=False),
        scratch_types=[pltpu.VMEM((8, 128), jnp.int32)]
                      + [pltpu.VMEM((128, F), jnp.float32)] * 8
                      + [pltpu.SemaphoreType.DMA, pltpu.SemaphoreType.DMA],
    )
    def k(tbl_hbm, idx_hbm, out_hbm, ib, r0, r1, r2, r3, r4, r5, r6, r7,
          semg, semw):
        c = lax.axis_index("c")
        s = lax.axis_index("s")
        wid = s * NC + c
        rs = [r0, r1, r2, r3, r4, r5, r6, r7]

        def body(q, carry):
            c0 = (wid * GPT + q) * 8
            pltpu.sync_copy(idx_hbm.at[pl.ds(c0, 8)], ib)

            @pl.when(q > 0)
            def _():
                for j in range(8):
                    pltpu.make_async_copy(out_hbm.at[pl.ds(0, 128)],
                                          rs[j], semw).wait()

            hs = [pltpu.async_copy(tbl_hbm.at[ib.at[j]], rs[j], semg)
                  for j in range(8)]
            for h in hs:
                h.wait()
            for j in range(8):
                pltpu.async_copy(rs[j], out_hbm.at[pl.ds((c0 + j) * 128, 128)],
                                 semw)
            return carry

        lax.fori_loop(0, GPT, body, 0)
        for j in range(8):
            pltpu.make_async_copy(out_hbm.at[pl.ds(0, 128)], rs[j], semw).wait()

    return k


def _make_scatter():
    mesh = plsc.VectorSubcoreMesh(core_axis_name="c", subcore_axis_name="s")
    base_n = NB_S // NS
    extra = NB_S - base_n * NS
    nfull = RPT // 128
    rem = RPT - nfull * 128

    npair = NB_S // 2
    base_p = npair // NS
    extra_p = npair - base_p * NS

    @functools.partial(
        pl.kernel,
        out_type=jax.ShapeDtypeStruct((2 * ACC_R, F), jnp.float32),
        mesh=mesh,
        compiler_params=pltpu.CompilerParams(use_tc_tiling_on_sc=False),
        scratch_types=[pltpu.VMEM((8, 128), jnp.int32),
                       pltpu.VMEM((128, F), jnp.float32),
                       pltpu.VMEM((128, F), jnp.float32)]
                      + [pltpu.SemaphoreType.DMA] * 3
                      + [pltpu.VMEM_SHARED((ACC_R, F), jnp.float32)],
    )
    def k(e_hbm, idx_hbm, zeros_hbm, out_hbm, ib, b0, b1,
          seme, sema, semz, acc):
        c = lax.axis_index("c")
        s = lax.axis_index("s")
        r0 = s * RPT
        pltpu.sync_copy(zeros_hbm, b0)
        hz = [pltpu.async_copy(b0, acc.at[pl.ds(r0 + j * 128, 128)], semz)
              for j in range(nfull)]
        hz.append(pltpu.async_copy(b0.at[pl.ds(0, rem)],
                                   acc.at[pl.ds(r0 + nfull * 128, rem)], semz))
        for h in hz:
            h.wait()
        plsc.subcore_barrier()

        np_t = base_p + jnp.where(s < extra_p, 1, 0)
        start = s * base_p + jnp.minimum(s, extra_p)

        def drain_adds():
            pltpu.make_async_copy(zeros_hbm, b0, sema).wait()
            pltpu.make_async_copy(zeros_hbm, b1, sema).wait()

        def do_pair(r, i0, i1):
            h0 = pltpu.async_copy(e_hbm.at[pl.ds(r * 128, 128)], b0, seme)
            h1 = pltpu.async_copy(e_hbm.at[pl.ds((r + 1) * 128, 128)], b1, semz)
            h0.wait()
            pltpu.async_copy(b0, acc.at[i0], sema, add=True)
            h1.wait()
            pltpu.async_copy(b1, acc.at[i1], sema, add=True)

        def super_body(u, carry):
            @pl.when(u > 0)
            def _():
                drain_adds()

            r0s = (start + u * 4) * 2
            pltpu.sync_copy(idx_hbm.at[pl.ds(c * NB_S + r0s, 8)], ib)
            for j in range(4):
                if j > 0:
                    drain_adds()
                do_pair(r0s + 2 * j, ib.at[2 * j], ib.at[2 * j + 1])
            return carry

        lax.fori_loop(0, 48, super_body, 0)

        def tail_body(kk, carry):
            drain_adds()
            r = (start + 192 + kk) * 2
            pltpu.sync_copy(idx_hbm.at[pl.ds(c * NB_S + r, 2)],
                            ib.at[pl.ds(0, 2)])
            do_pair(r, ib.at[0], ib.at[1])
            return carry

        lax.fori_loop(0, np_t - 192, tail_body, 0)
        drain_adds()
        plsc.subcore_barrier()

        for g in range(6):
            h0 = pltpu.async_copy(acc.at[pl.ds(r0 + (2 * g) * 128, 128)],
                                  b0, seme)
            h1 = pltpu.async_copy(acc.at[pl.ds(r0 + (2 * g + 1) * 128, 128)],
                                  b1, semz)
            h0.wait()
            w0 = pltpu.async_copy(
                b0, out_hbm.at[pl.ds(c * ACC_R + r0 + (2 * g) * 128, 128)],
                sema)
            h1.wait()
            w1 = pltpu.async_copy(
                b1, out_hbm.at[pl.ds(c * ACC_R + r0 + (2 * g + 1) * 128, 128)],
                sema)
            w0.wait()
            w1.wait()
        rr = r0 + nfull * 128
        pltpu.sync_copy(acc.at[pl.ds(rr, rem)], b0.at[pl.ds(0, rem)])
        pltpu.sync_copy(b0.at[pl.ds(0, rem)],
                        out_hbm.at[pl.ds(c * ACC_R + rr, rem)])

    return k


_gather_nodes = _make_gather(N)
_gather_enc = _make_gather(2 * N)
_scatter = _make_scatter()


def kernel(x, graph, ne_W1, ne_b1, ne_W2, ne_b2, ee_W1, ee_b1, ee_W2, ee_b2,
           cn_W1, cn_b1, cn_W2, cn_b2, ce_W1, ce_b1, ce_W2, ce_b2,
           out_W1, out_b1, out_W2, out_b2):
    src = graph[0]
    dst = graph[1]

    x8 = jnp.pad(x, ((0, 0), (0, 5)))
    nw1p = jnp.pad(ne_W1, ((0, 5), (0, 0)))
    wa = jnp.pad(ee_W1[:3], ((0, 5), (0, 0)))
    wb = jnp.pad(ee_W1[3:], ((0, 5), (0, 0)))
    r1 = lambda b: b.reshape(1, -1)

    pad_g = ((0, NB_GP - NB_G), (0, 0))
    idx_enc = jnp.pad(jnp.concatenate([src, dst + N]).reshape(NB_G, 128), pad_g)
    idx_it = jnp.pad(jnp.concatenate([src, dst]).reshape(NB_G, 128), pad_g)
    loc0 = jnp.where(dst < HALF, dst, TRASH)
    loc1 = jnp.where(dst >= HALF, dst - HALF, TRASH)
    idx_sc = jnp.concatenate([loc0, loc1]).reshape(2 * NB_S, 128)
    zeros128 = jnp.zeros((128, F), jnp.float32)

    w1a, w1b = cn_W1[:F], cn_W1[F:]
    ewa, ewb, ewc = ce_W1[:F], ce_W1[F:2 * F], ce_W1[2 * F:]
    w2p = jnp.pad(out_W2, ((0, 0), (0, 128 - 12)))
    b2p = jnp.pad(out_b2, (0, 128 - 12)).reshape(1, 128)

    nodes, xa, xb = _enc_pre(x8, nw1p, r1(ne_b1), ne_W2, r1(ne_b2), wa, wb)
    g0 = _gather_enc(jnp.concatenate([xa, xb], axis=0), idx_enc)
    edges = _edge_enc(g0, ee_W2, r1(ee_b1), r1(ee_b2))

    for _ in range(ITERS):
        msg2 = _scatter(edges, idx_sc, zeros128).reshape(2, ACC_R, F)
        nodes = _node_update(nodes, msg2, w1a, w1b, r1(cn_b1), cn_W2, r1(cn_b2))
        g = _gather_nodes(nodes, idx_it)
        edges = _edge_update(g, edges, ewa, ewb, ewc, r1(ce_b1), ce_W2, r1(ce_b2))

    embp = _head(nodes, out_W1, r1(out_b1), w2p, b2p)
    return (embp[:, :12], nodes, edges)


# speedup vs baseline: 2.4770x; 1.2223x over previous
import functools

import jax
import jax.numpy as jnp
from jax import lax
from jax.experimental import pallas as pl
from jax.experimental.pallas import tpu as pltpu
from jax.experimental.pallas import tpu_sc as plsc

N = 50000
E = 800000
ITERS = 8
F = 64
NC, NS = 2, 16
NW = NC * NS
HALF = N // 2
TRASH = HALF
ACC_R = 25024
RPT = ACC_R // NS
NB_G = (2 * E) // 128
NB_S = E // 128


def _silu(x):
    return x / (1.0 + jnp.exp(-x))


def _full(shape):
    return pl.BlockSpec(shape, lambda i: (0,) * len(shape))


def _enc_pre(x8, nw1, nb1, nw2, nb2, wa, wb, eb1):
    B = 10000

    def body(x_ref, nw1_r, nb1_r, nw2_r, nb2_r, wa_r, wb_r, eb1_r,
             n_out, a_out, b_out):
        xv = x_ref[...]
        h = _silu(xv @ nw1_r[...] + nb1_r[...])
        n_out[...] = _silu(h @ nw2_r[...] + nb2_r[...])
        a_out[...] = xv @ wa_r[...] + eb1_r[...]
        b_out[...] = xv @ wb_r[...]

    return pl.pallas_call(
        body,
        grid=(N // B,),
        in_specs=[pl.BlockSpec((B, 8), lambda i: (i, 0)),
                  _full((8, F)), _full((1, F)), _full((F, F)), _full((1, F)),
                  _full((8, F)), _full((8, F)), _full((1, F))],
        out_specs=[pl.BlockSpec((B, F), lambda i: (i, 0))] * 3,
        out_shape=[jax.ShapeDtypeStruct((N, F), jnp.float32)] * 3,
    )(x8, nw1, nb1, nw2, nb2, wa, wb, eb1)


def _edge_enc(g, w2, b2):
    B = 8000
    nblk = E // B

    def body(g_ref, w2_r, b2_r, out):
        h = _silu(g_ref[...])
        out[...] = _silu(h @ w2_r[...] + b2_r[...])

    return pl.pallas_call(
        body,
        grid=(nblk,),
        in_specs=[pl.BlockSpec((B, F), lambda i: (i, 0)),
                  _full((F, F)), _full((1, F))],
        out_specs=pl.BlockSpec((B, F), lambda i: (i, 0)),
        out_shape=jax.ShapeDtypeStruct((E, F), jnp.float32),
    )(g, w2, b2)


def _node_update(nodes, msg2, w1a, w1b, b1, w2, b2, ewa, ewb, eb1):
    B = 5000
    nhalf = HALF // B

    def body(n_ref, m_ref, w1a_r, w1b_r, b1_r, w2_r, b2_r, ewa_r, ewb_r,
             eb1_r, out, a_out, b_out):
        n = n_ref[...]
        m = m_ref[0]
        h = _silu(n @ w1a_r[...] + m @ w1b_r[...] + b1_r[...])
        n2 = _silu(h @ w2_r[...] + b2_r[...]) + n
        out[...] = n2
        a_out[...] = n2 @ ewa_r[...] + eb1_r[...]
        b_out[...] = n2 @ ewb_r[...]

    return pl.pallas_call(
        body,
        grid=(N // B,),
        in_specs=[pl.BlockSpec((B, F), lambda i: (i, 0)),
                  pl.BlockSpec((1, B, F), lambda i: (i // nhalf, i % nhalf, 0)),
                  _full((F, F)), _full((F, F)), _full((1, F)),
                  _full((F, F)), _full((1, F)),
                  _full((F, F)), _full((F, F)), _full((1, F))],
        out_specs=[pl.BlockSpec((B, F), lambda i: (i, 0))] * 3,
        out_shape=[jax.ShapeDtypeStruct((N, F), jnp.float32)] * 3,
    )(nodes, msg2, w1a, w1b, b1, w2, b2, ewa, ewb, eb1)


def _edge_update(g, edges, wc, w2, b2):
    B = 8000
    nblk = E // B

    def body(g_ref, e_ref, wc_r, w2_r, b2_r, out):
        e = e_ref[...]
        h = _silu(g_ref[...] + e @ wc_r[...])
        out[...] = jnp.tanh(h @ w2_r[...] + b2_r[...]) + e

    return pl.pallas_call(
        body,
        grid=(nblk,),
        in_specs=[pl.BlockSpec((B, F), lambda i: (i, 0)),
                  pl.BlockSpec((B, F), lambda i: (i, 0)),
                  _full((F, F)), _full((F, F)), _full((1, F))],
        out_specs=pl.BlockSpec((B, F), lambda i: (i, 0)),
        out_shape=jax.ShapeDtypeStruct((E, F), jnp.float32),
    )(g, edges, wc, w2, b2)


def _head(nodes, w1, b1, w2p, b2p):
    B = 10000

    def body(n_ref, w1_r, b1_r, w2_r, b2_r, out):
        h = _silu(n_ref[...] @ w1_r[...] + b1_r[...])
        e = h @ w2_r[...] + b2_r[...]
        nrm = jnp.sqrt(jnp.sum(e * e, axis=1, keepdims=True))
        out[...] = e / jnp.maximum(nrm, 1e-12)

    return pl.pallas_call(
        body,
        grid=(N // B,),
        in_specs=[pl.BlockSpec((B, F), lambda i: (i, 0)),
                  _full((F, F)), _full((1, F)), _full((F, 128)), _full((1, 128))],
        out_specs=pl.BlockSpec((B, 128), lambda i: (i, 0)),
        out_shape=jax.ShapeDtypeStruct((N, 128), jnp.float32),
    )(nodes, w1, b1, w2p, b2p)


NB_SP = 6272
GPT = NB_SP // 4 // NW


def _make_gather_sum():
    mesh = plsc.VectorSubcoreMesh(core_axis_name="c", subcore_axis_name="s")

    @functools.partial(
        pl.kernel,
        out_type=jax.ShapeDtypeStruct((NB_SP * 128, F), jnp.float32),
        mesh=mesh,
        compiler_params=pltpu.CompilerParams(use_tc_tiling_on_sc=False),
        scratch_types=[pltpu.VMEM((4, 128), jnp.int32),
                       pltpu.VMEM((4, 128), jnp.int32)]
                      + [pltpu.VMEM((128, F), jnp.float32)] * 4
                      + [pltpu.SemaphoreType.DMA, pltpu.SemaphoreType.DMA],
    )
    def k(tbla_hbm, tblb_hbm, idxs_hbm, idxd_hbm, out_hbm,
          ibs, ibd, r0, r1, r2, r3, semg, semw):
        c = lax.axis_index("c")
        s = lax.axis_index("s")
        wid = s * NC + c
        rs = [r0, r1, r2, r3]

        def body(q, carry):
            c0 = (wid * GPT + q) * 4
            pltpu.sync_copy(idxs_hbm.at[pl.ds(c0, 4)], ibs)
            pltpu.sync_copy(idxd_hbm.at[pl.ds(c0, 4)], ibd)

            @pl.when(q > 0)
            def _():
                for j in range(4):
                    pltpu.make_async_copy(out_hbm.at[pl.ds(0, 128)],
                                          rs[j], semw).wait()

            ha = [pltpu.async_copy(tbla_hbm.at[ibs.at[j]], rs[j], semg)
                  for j in range(4)]
            for h in ha:
                h.wait()
            hb = [pltpu.async_copy(tblb_hbm.at[ibd.at[j]], rs[j], semg,
                                   add=True) for j in range(4)]
            for h in hb:
                h.wait()
            for j in range(4):
                pltpu.async_copy(rs[j], out_hbm.at[pl.ds((c0 + j) * 128, 128)],
                                 semw)
            return carry

        lax.fori_loop(0, GPT, body, 0)
        for j in range(4):
            pltpu.make_async_copy(out_hbm.at[pl.ds(0, 128)], rs[j], semw).wait()

    return k


def _make_scatter():
    mesh = plsc.VectorSubcoreMesh(core_axis_name="c", subcore_axis_name="s")
    base_n = NB_S // NS
    extra = NB_S - base_n * NS
    nfull = RPT // 128
    rem = RPT - nfull * 128

    npair = NB_S // 2
    base_p = npair // NS
    extra_p = npair - base_p * NS

    @functools.partial(
        pl.kernel,
        out_type=jax.ShapeDtypeStruct((2 * ACC_R, F), jnp.float32),
        mesh=mesh,
        compiler_params=pltpu.CompilerParams(use_tc_tiling_on_sc=False),
        scratch_types=[pltpu.VMEM((8, 128), jnp.int32),
                       pltpu.VMEM((128, F), jnp.float32),
                       pltpu.VMEM((128, F), jnp.float32)]
                      + [pltpu.SemaphoreType.DMA] * 3
                      + [pltpu.VMEM_SHARED((ACC_R, F), jnp.float32)],
    )
    def k(e_hbm, idx_hbm, zeros_hbm, out_hbm, ib, b0, b1,
          seme, sema, semz, acc):
        c = lax.axis_index("c")
        s = lax.axis_index("s")
        r0 = s * RPT
        pltpu.sync_copy(zeros_hbm, b0)
        hz = [pltpu.async_copy(b0, acc.at[pl.ds(r0 + j * 128, 128)], semz)
              for j in range(nfull)]
        hz.append(pltpu.async_copy(b0.at[pl.ds(0, rem)],
                                   acc.at[pl.ds(r0 + nfull * 128, rem)], semz))
        for h in hz:
            h.wait()
        plsc.subcore_barrier()

        np_t = base_p + jnp.where(s < extra_p, 1, 0)
        start = s * base_p + jnp.minimum(s, extra_p)

        def drain_adds():
            pltpu.make_async_copy(zeros_hbm, b0, sema).wait()
            pltpu.make_async_copy(zeros_hbm, b1, sema).wait()

        def do_pair(r, i0, i1):
            h0 = pltpu.async_copy(e_hbm.at[pl.ds(r * 128, 128)], b0, seme)
            h1 = pltpu.async_copy(e_hbm.at[pl.ds((r + 1) * 128, 128)], b1, semz)
            h0.wait()
            pltpu.async_copy(b0, acc.at[i0], sema, add=True)
            h1.wait()
            pltpu.async_copy(b1, acc.at[i1], sema, add=True)

        def super_body(u, carry):
            @pl.when(u > 0)
            def _():
                drain_adds()

            r0s = (start + u * 4) * 2
            pltpu.sync_copy(idx_hbm.at[pl.ds(c * NB_S + r0s, 8)], ib)
            for j in range(4):
                if j > 0:
                    drain_adds()
                do_pair(r0s + 2 * j, ib.at[2 * j], ib.at[2 * j + 1])
            return carry

        lax.fori_loop(0, 48, super_body, 0)

        def tail_body(kk, carry):
            drain_adds()
            r = (start + 192 + kk) * 2
            pltpu.sync_copy(idx_hbm.at[pl.ds(c * NB_S + r, 2)],
                            ib.at[pl.ds(0, 2)])
            do_pair(r, ib.at[0], ib.at[1])
            return carry

        lax.fori_loop(0, np_t - 192, tail_body, 0)
        drain_adds()
        plsc.subcore_barrier()

        for g in range(6):
            h0 = pltpu.async_copy(acc.at[pl.ds(r0 + (2 * g) * 128, 128)],
                                  b0, seme)
            h1 = pltpu.async_copy(acc.at[pl.ds(r0 + (2 * g + 1) * 128, 128)],
                                  b1, semz)
            h0.wait()
            w0 = pltpu.async_copy(
                b0, out_hbm.at[pl.ds(c * ACC_R + r0 + (2 * g) * 128, 128)],
                sema)
            h1.wait()
            w1 = pltpu.async_copy(
                b1, out_hbm.at[pl.ds(c * ACC_R + r0 + (2 * g + 1) * 128, 128)],
                sema)
            w0.wait()
            w1.wait()
        rr = r0 + nfull * 128
        pltpu.sync_copy(acc.at[pl.ds(rr, rem)], b0.at[pl.ds(0, rem)])
        pltpu.sync_copy(b0.at[pl.ds(0, rem)],
                        out_hbm.at[pl.ds(c * ACC_R + rr, rem)])

    return k


_gather_sum = _make_gather_sum()
_scatter = _make_scatter()


def kernel(x, graph, ne_W1, ne_b1, ne_W2, ne_b2, ee_W1, ee_b1, ee_W2, ee_b2,
           cn_W1, cn_b1, cn_W2, cn_b2, ce_W1, ce_b1, ce_W2, ce_b2,
           out_W1, out_b1, out_W2, out_b2):
    src = graph[0]
    dst = graph[1]

    x8 = jnp.pad(x, ((0, 0), (0, 5)))
    nw1p = jnp.pad(ne_W1, ((0, 5), (0, 0)))
    wa = jnp.pad(ee_W1[:3], ((0, 5), (0, 0)))
    wb = jnp.pad(ee_W1[3:], ((0, 5), (0, 0)))
    r1 = lambda b: b.reshape(1, -1)

    pad_s = ((0, NB_SP - NB_S), (0, 0))
    idx_src = jnp.pad(src.reshape(NB_S, 128), pad_s)
    idx_dst = jnp.pad(dst.reshape(NB_S, 128), pad_s)
    loc0 = jnp.where(dst < HALF, dst, TRASH)
    loc1 = jnp.where(dst >= HALF, dst - HALF, TRASH)
    idx_sc = jnp.concatenate([loc0, loc1]).reshape(2 * NB_S, 128)
    zeros128 = jnp.zeros((128, F), jnp.float32)

    w1a, w1b = cn_W1[:F], cn_W1[F:]
    ewa, ewb, ewc = ce_W1[:F], ce_W1[F:2 * F], ce_W1[2 * F:]
    w2p = jnp.pad(out_W2, ((0, 0), (0, 128 - 12)))
    b2p = jnp.pad(out_b2, (0, 128 - 12)).reshape(1, 128)

    nodes, xa, xb = _enc_pre(x8, nw1p, r1(ne_b1), ne_W2, r1(ne_b2), wa, wb,
                             r1(ee_b1))
    g0 = _gather_sum(xa, xb, idx_src, idx_dst)
    edges = _edge_enc(g0, ee_W2, r1(ee_b2))

    for _ in range(ITERS):
        msg2 = _scatter(edges, idx_sc, zeros128).reshape(2, ACC_R, F)
        nodes, ta, tb = _node_update(nodes, msg2, w1a, w1b, r1(cn_b1), cn_W2,
                                     r1(cn_b2), ewa, ewb, r1(ce_b1))
        g = _gather_sum(ta, tb, idx_src, idx_dst)
        edges = _edge_update(g, edges, ewc, ce_W2, r1(ce_b2))

    embp = _head(nodes, out_W1, r1(out_b1), w2p, b2p)
    return (embp[:, :12], nodes, edges)
